# Initial kernel scaffold; baseline (speedup 1.0000x reference)
import functools

import jax
import jax.numpy as jnp
from jax import lax
from jax.experimental import pallas as pl
from jax.experimental.pallas import tpu as pltpu
from jax.experimental.pallas import tpu_sc as plsc

H = 64
LANES = 16
C = 128


@functools.lru_cache(maxsize=None)
def _make_kernel(N, num_cores, num_subcores):
    NW = num_cores * num_subcores
    R = N // NW
    n_chunks = R // C
    assert R % C == 0 and N % NW == 0

    mesh = plsc.VectorSubcoreMesh(core_axis_name="c", subcore_axis_name="s")

    @functools.partial(
        pl.kernel,
        mesh=mesh,
        out_type=jax.ShapeDtypeStruct((N, H), jnp.float32),
        scratch_types=[
            pltpu.VMEM((C,), jnp.int32),
            pltpu.VMEM((C,), jnp.int32),
            pltpu.VMEM((C,), jnp.int32),
            pltpu.VMEM((C,), jnp.int32),
            pltpu.VMEM((C, H), jnp.float32),
            pltpu.VMEM((C, H), jnp.float32),
            pltpu.VMEM((C, H), jnp.float32),
            pltpu.SemaphoreType.DMA,
            pltpu.SemaphoreType.DMA,
            pltpu.SemaphoreType.DMA,
        ],
    )
    def k(x_hbm, pid_hbm, act_hbm, pos_hbm, ptab_hbm, ctab_hbm, out_hbm,
          pid_v, act_v, pos_v, idx2_v, x_v, prow_v, crow_v,
          sem_p, sem_c, sem_x):
        wid = lax.axis_index("s") * num_cores + lax.axis_index("c")
        base = wid * R

        def chunk_body(i, carry):
            off = base + i * C
            pltpu.sync_copy(pid_hbm.at[pl.ds(off, C)], pid_v)
            pltpu.sync_copy(act_hbm.at[pl.ds(off, C)], act_v)
            pltpu.sync_copy(pos_hbm.at[pl.ds(off, C)], pos_v)
            for g in range(C // LANES):
                sl = pl.ds(g * LANES, LANES)
                idx2_v[sl] = act_v[sl] * 10 + pos_v[sl]
            cp_x = pltpu.async_copy(x_hbm.at[pl.ds(off, C)], x_v, sem_x)
            cp_p = pltpu.async_copy(ptab_hbm.at[pid_v], prow_v, sem_p)
            cp_c = pltpu.async_copy(ctab_hbm.at[idx2_v], crow_v, sem_c)
            cp_x.wait()
            cp_p.wait()
            cp_c.wait()

            def row_body(r, rc):
                for g in range(H // LANES):
                    sl = pl.ds(g * LANES, LANES)
                    x_v[r, sl] = x_v[r, sl] + prow_v[r, sl] + crow_v[r, sl]
                return rc

            lax.fori_loop(0, C, row_body, 0)
            pltpu.sync_copy(x_v, out_hbm.at[pl.ds(off, C)])
            return carry

        lax.fori_loop(0, n_chunks, chunk_body, 0)

    return k


def kernel(x, player_ids, actions, positions, player_weight, action_weight,
           position_weight):
    B, L, Hd = x.shape
    N = B * L
    xf = x.reshape(N, Hd)
    pid = player_ids.reshape(N).astype(jnp.int32)
    act = actions.reshape(N).astype(jnp.int32)
    pos = positions.reshape(N).astype(jnp.int32)
    combo = (action_weight[:, None, :] + position_weight[None, :, :]).reshape(
        -1, Hd)
    info = plsc.get_sparse_core_info()
    out = _make_kernel(N, info.num_cores, info.num_subcores)(
        xf, pid, act, pos, player_weight, combo)
    return out.reshape(B, L, Hd)


# speedup vs baseline: 1.9722x; 1.9722x over previous
import functools

import jax
import jax.numpy as jnp
from jax import lax
from jax.experimental import pallas as pl
from jax.experimental.pallas import tpu as pltpu
from jax.experimental.pallas import tpu_sc as plsc

H = 64
LANES = 16
C = 128


@functools.lru_cache(maxsize=None)
def _make_kernel(N, num_cores, num_subcores):
    NW = num_cores * num_subcores
    R = N // NW
    n_chunks = R // C
    assert R % C == 0 and N % NW == 0

    mesh = plsc.VectorSubcoreMesh(core_axis_name="c", subcore_axis_name="s")

    @functools.partial(
        pl.kernel,
        mesh=mesh,
        compiler_params=pltpu.CompilerParams(use_tc_tiling_on_sc=False),
        out_type=jax.ShapeDtypeStruct((N, H), jnp.float32),
        scratch_types=[
            pltpu.VMEM((C,), jnp.int32),
            pltpu.VMEM((C,), jnp.int32),
            pltpu.VMEM((C,), jnp.int32),
            pltpu.VMEM((C,), jnp.int32),
            pltpu.VMEM((C, H), jnp.float32),
            pltpu.VMEM((C, H), jnp.float32),
            pltpu.VMEM((C, H), jnp.float32),
            pltpu.SemaphoreType.DMA,
            pltpu.SemaphoreType.DMA,
            pltpu.SemaphoreType.DMA,
        ],
    )
    def k(x_hbm, pid_hbm, act_hbm, pos_hbm, ptab_hbm, ctab_hbm, out_hbm,
          pid_v, act_v, pos_v, idx2_v, x_v, prow_v, crow_v,
          sem_p, sem_c, sem_x):
        wid = lax.axis_index("s") * num_cores + lax.axis_index("c")
        base = wid * R

        def chunk_body(i, carry):
            off = base + i * C
            pltpu.sync_copy(pid_hbm.at[pl.ds(off, C)], pid_v)
            pltpu.sync_copy(act_hbm.at[pl.ds(off, C)], act_v)
            pltpu.sync_copy(pos_hbm.at[pl.ds(off, C)], pos_v)
            for g in range(C // LANES):
                sl = pl.ds(g * LANES, LANES)
                idx2_v[sl] = act_v[sl] * 10 + pos_v[sl]
            cp_x = pltpu.async_copy(x_hbm.at[pl.ds(off, C)], x_v, sem_x)
            cp_p = pltpu.async_copy(ptab_hbm.at[pid_v], prow_v, sem_p)
            cp_c = pltpu.async_copy(ctab_hbm.at[idx2_v], crow_v, sem_c)
            cp_x.wait()
            cp_p.wait()
            cp_c.wait()

            def row_body(r, rc):
                for g in range(H // LANES):
                    sl = pl.ds(g * LANES, LANES)
                    x_v[r, sl] = x_v[r, sl] + prow_v[r, sl] + crow_v[r, sl]
                return rc

            lax.fori_loop(0, C, row_body, 0)
            pltpu.sync_copy(x_v, out_hbm.at[pl.ds(off, C)])
            return carry

        lax.fori_loop(0, n_chunks, chunk_body, 0)

    return k


def kernel(x, player_ids, actions, positions, player_weight, action_weight,
           position_weight):
    B, L, Hd = x.shape
    N = B * L
    xf = x.reshape(N, Hd)
    pid = player_ids.reshape(N).astype(jnp.int32)
    act = actions.reshape(N).astype(jnp.int32)
    pos = positions.reshape(N).astype(jnp.int32)
    combo = (action_weight[:, None, :] + position_weight[None, :, :]).reshape(
        -1, Hd)
    info = plsc.get_sparse_core_info()
    out = _make_kernel(N, info.num_cores, info.num_subcores)(
        xf, pid, act, pos, player_weight, combo)
    return out.reshape(B, L, Hd)


# speedup vs baseline: 1.9867x; 1.0074x over previous
import functools

import jax
import jax.numpy as jnp
from jax import lax
from jax.experimental import pallas as pl
from jax.experimental.pallas import tpu as pltpu
from jax.experimental.pallas import tpu_sc as plsc

H = 64
LANES = 16
C = 128
NBUF = 2


@functools.lru_cache(maxsize=None)
def _make_kernel(N, num_cores, num_subcores):
    NW = num_cores * num_subcores
    R = N // NW
    nch = R // C
    assert N % NW == 0 and R % C == 0 and nch % NBUF == 0

    mesh = plsc.VectorSubcoreMesh(core_axis_name="c", subcore_axis_name="s")

    data_bufs = []
    for _ in range(NBUF):
        data_bufs += [
            pltpu.VMEM((C, H), jnp.float32),
            pltpu.VMEM((C, H), jnp.float32),
            pltpu.VMEM((C, H), jnp.float32),
            pltpu.VMEM((C, H), jnp.float32),
            pltpu.SemaphoreType.DMA,
            pltpu.SemaphoreType.DMA,
        ]

    @functools.partial(
        pl.kernel,
        mesh=mesh,
        compiler_params=pltpu.CompilerParams(use_tc_tiling_on_sc=False),
        out_type=jax.ShapeDtypeStruct((N, H), jnp.float32),
        scratch_types=[
            pltpu.VMEM((nch, C), jnp.int32),
            pltpu.VMEM((nch, C), jnp.int32),
            pltpu.VMEM((nch, C), jnp.int32),
        ] + data_bufs,
    )
    def k(x_hbm, pid_hbm, act_hbm, pos_hbm, ptab_hbm, ctab_hbm, out_hbm,
          pid_all, act_all, idx2_all, *bufs):
        xb = [bufs[6 * b + 0] for b in range(NBUF)]
        pb = [bufs[6 * b + 1] for b in range(NBUF)]
        cb = [bufs[6 * b + 2] for b in range(NBUF)]
        ob = [bufs[6 * b + 3] for b in range(NBUF)]
        isem = [bufs[6 * b + 4] for b in range(NBUF)]
        osem = [bufs[6 * b + 5] for b in range(NBUF)]

        wid = lax.axis_index("s") * num_cores + lax.axis_index("c")
        base = wid * R

        pltpu.sync_copy(pid_hbm.at[pl.ds(wid * nch, nch)], pid_all)
        pltpu.sync_copy(act_hbm.at[pl.ds(wid * nch, nch)], act_all)
        pltpu.sync_copy(pos_hbm.at[pl.ds(wid * nch, nch)], idx2_all)

        def fuse_body(j, carry):
            for g in range(C // LANES):
                sl = pl.ds(g * LANES, LANES)
                idx2_all[j, sl] = act_all[j, sl] * 10 + idx2_all[j, sl]
            return carry

        lax.fori_loop(0, nch, fuse_body, 0)

        def issue_in(i, p):
            off = base + i * C
            pltpu.async_copy(x_hbm.at[pl.ds(off, C)], xb[p], isem[p])
            pltpu.async_copy(ptab_hbm.at[pid_all.at[i]], pb[p], isem[p])
            pltpu.async_copy(ctab_hbm.at[idx2_all.at[i]], cb[p], isem[p])

        def wait_in(i, p):
            off = base + i * C
            pltpu.make_async_copy(x_hbm.at[pl.ds(off, C)], xb[p], isem[p]).wait()
            pltpu.make_async_copy(ptab_hbm.at[pid_all.at[i]], pb[p], isem[p]).wait()
            pltpu.make_async_copy(ctab_hbm.at[idx2_all.at[i]], cb[p], isem[p]).wait()

        def wait_out(p):
            pltpu.make_async_copy(ob[p], out_hbm.at[pl.ds(base, C)], osem[p]).wait()

        for p in range(NBUF):
            issue_in(p, p)

        def step(t, carry):
            for s in range(NBUF):
                i = NBUF * t + s
                p = s
                wait_in(i, p)

                @pl.when(t > 0)
                def _():
                    wait_out(p)

                def row_body(r, rc):
                    for g in range(H // LANES):
                        sl = pl.ds(g * LANES, LANES)
                        ob[p][r, sl] = xb[p][r, sl] + pb[p][r, sl] + cb[p][r, sl]
                    return rc

                lax.fori_loop(0, C, row_body, 0)
                pltpu.async_copy(ob[p], out_hbm.at[pl.ds(base + i * C, C)],
                                 osem[p])

                @pl.when(i + NBUF < nch)
                def _():
                    issue_in(i + NBUF, p)
            return carry

        lax.fori_loop(0, nch // NBUF, step, 0)
        for p in range(NBUF):
            wait_out(p)

    return k


def kernel(x, player_ids, actions, positions, player_weight, action_weight,
           position_weight):
    B, L, Hd = x.shape
    N = B * L
    xf = x.reshape(N, Hd)
    pid = player_ids.reshape(N // C, C).astype(jnp.int32)
    act = actions.reshape(N // C, C).astype(jnp.int32)
    pos = positions.reshape(N // C, C).astype(jnp.int32)
    combo = (action_weight[:, None, :] + position_weight[None, :, :]).reshape(
        -1, Hd)
    info = plsc.get_sparse_core_info()
    out = _make_kernel(N, info.num_cores, info.num_subcores)(
        xf, pid, act, pos, player_weight, combo)
    return out.reshape(B, L, Hd)
